# baseline (speedup 1.0000x reference)
import functools

import jax
import jax.numpy as jnp
from jax import lax
from jax.experimental import pallas as pl
from jax.experimental.pallas import tpu as pltpu
from jax.experimental.pallas import tpu_sc as plsc

_L = 12
_P = 100
_LEN = 5
_D = 768
_K = 5
_B = 32
_S = 2048

_ROW = _LEN * _D
_ROWS = _L * _B * _K
_NEG = -3.0e38


def _tc_body(x_ref, sk_ref, tk_ref,
             s_sim_ref, t_sim_ref, s_idx_ref, t_idx_ref,
             s_red_ref, t_red_ref):
    b = pl.program_id(0)
    xb = x_ref[0]
    mean = jnp.sum(xb, axis=0, keepdims=True) * (1.0 / _S)
    n2 = jnp.sum(mean * mean, axis=1, keepdims=True)
    xn = mean * lax.rsqrt(jnp.maximum(n2, 1e-12))

    @pl.when(b == 0)
    def _():
        s_red_ref[...] = jnp.zeros((1, 1), jnp.float32)
        t_red_ref[...] = jnp.zeros((1, 1), jnp.float32)

    ii = lax.broadcasted_iota(jnp.int32, (1, _P), 1)
    ii5 = lax.broadcasted_iota(jnp.int32, (1, _K), 1)

    def route(k_ref, sim_ref, idx_ref, red_ref):
        kk = k_ref[...]
        kn2 = jnp.sum(kk * kk, axis=1, keepdims=True)
        kn = kk * lax.rsqrt(jnp.maximum(kn2, 1e-12))
        sim = lax.dot_general(xn, kn, (((1,), (1,)), ((), ())),
                              preferred_element_type=jnp.float32)
        sim_ref[0] = sim
        row = sim
        racc = jnp.zeros((1, 1), jnp.float32)
        ivec = jnp.zeros((1, _K), jnp.int32)
        for k in range(_K):
            mx = jnp.max(row, axis=1, keepdims=True)
            am = jnp.min(jnp.where(row == mx, ii, _P),
                         axis=1, keepdims=True)
            ivec = jnp.where(ii5 == k, am, ivec)
            racc = racc + mx
            row = jnp.where(ii == am, _NEG, row)
        idx_ref[0] = ivec
        red_ref[...] = red_ref[...] + racc

        @pl.when(b == _B - 1)
        def _():
            red_ref[...] = red_ref[...] * (1.0 / _B)

    route(sk_ref, s_sim_ref, s_idx_ref, s_red_ref)
    route(tk_ref, t_sim_ref, t_idx_ref, t_red_ref)


_route_call = pl.pallas_call(
    _tc_body,
    grid=(_B,),
    in_specs=[
        pl.BlockSpec((1, _S, _D), lambda b: (b, 0, 0)),
        pl.BlockSpec((_P, _D), lambda b: (0, 0)),
        pl.BlockSpec((_P, _D), lambda b: (0, 0)),
    ],
    out_specs=[
        pl.BlockSpec((1, 1, _P), lambda b: (b, 0, 0)),
        pl.BlockSpec((1, 1, _P), lambda b: (b, 0, 0)),
        pl.BlockSpec((1, 1, _K), lambda b: (b, 0, 0)),
        pl.BlockSpec((1, 1, _K), lambda b: (b, 0, 0)),
        pl.BlockSpec((1, 1), lambda b: (0, 0)),
        pl.BlockSpec((1, 1), lambda b: (0, 0)),
    ],
    out_shape=[
        jax.ShapeDtypeStruct((_B, 1, _P), jnp.float32),
        jax.ShapeDtypeStruct((_B, 1, _P), jnp.float32),
        jax.ShapeDtypeStruct((_B, 1, _K), jnp.int32),
        jax.ShapeDtypeStruct((_B, 1, _K), jnp.int32),
        jax.ShapeDtypeStruct((1, 1), jnp.float32),
        jax.ShapeDtypeStruct((1, 1), jnp.float32),
    ],
)

_NC = 2
_NS = 16
_NW = _NC * _NS
_CH = 16
_NCHUNKS = _ROWS // _CH
_CPL = (_B * _K) // _CH
@functools.cache
def _make_sc_gather():
    mesh = plsc.VectorSubcoreMesh(core_axis_name="c", subcore_axis_name="s",
                                  num_cores=_NC, num_subcores=_NS)

    @functools.partial(
        pl.kernel, mesh=mesh,
        out_type=[jax.ShapeDtypeStruct((_ROWS, _ROW), jnp.float32),
                  jax.ShapeDtypeStruct((_ROWS, _ROW), jnp.float32)],
        scratch_types=[
            pltpu.VMEM((_B * _K,), jnp.int32),
            pltpu.VMEM((_B * _K,), jnp.int32),
            pltpu.VMEM((_CH, _ROW), jnp.float32),
            pltpu.VMEM((_CH, _ROW), jnp.float32),
            pltpu.SemaphoreType.DMA,
            pltpu.SemaphoreType.DMA,
        ],
    )
    def _sc_gather(s_tab, t_tab, s_idx, t_idx, s_out, t_out,
                   s_idx_v, t_idx_v, buf_a, buf_b, sem_a, sem_b):
        wid = lax.axis_index("s") * _NC + lax.axis_index("c")
        pltpu.sync_copy(s_idx, s_idx_v)
        pltpu.sync_copy(t_idx, t_idx_v)

        bufs = (buf_a, buf_b)
        sems = (sem_a, sem_b)
        iot = lax.iota(jnp.int32, _CH)

        def run(tab, idx_v, out):
            def step(c, buf, sem):
                lyr = c // _CPL
                rem0 = (c % _CPL) * _CH
                flat = lyr * _P + idx_v[pl.ds(rem0, _CH)]
                pltpu.async_copy(tab.at[flat], buf, sem).wait()
                pltpu.sync_copy(buf, out.at[pl.ds(c * _CH, _CH)])

            for j in range(_NCHUNKS // _NW):
                step(j * _NW + wid, bufs[j % 2], sems[j % 2])

            @pl.when(wid < _NCHUNKS % _NW)
            def _():
                step((_NCHUNKS // _NW) * _NW + wid, bufs[0], sems[0])

        run(s_tab, s_idx_v, s_out)
        run(t_tab, t_idx_v, t_out)

    return _sc_gather


def kernel(x_embed, s_prompt, t_prompt, s_prompt_key, t_prompt_key):
    (s_sim3, t_sim3, s_idx3, t_idx3, s_red, t_red) = _route_call(
        x_embed, s_prompt_key, t_prompt_key)
    s_idx = s_idx3.reshape(_B, _K)
    t_idx = t_idx3.reshape(_B, _K)
    s_flat, t_flat = _make_sc_gather()(
        s_prompt.reshape(_L * _P, _ROW),
        t_prompt.reshape(_L * _P, _ROW),
        s_idx.reshape(_B * _K),
        t_idx.reshape(_B * _K))
    return (s_sim3.reshape(_B, _P), t_sim3.reshape(_B, _P), s_idx, t_idx,
            s_flat.reshape(_L, _B, _K * _LEN, _D),
            t_flat.reshape(_L, _B, _K * _LEN, _D),
            s_red.reshape(()), t_red.reshape(()))


# speedup vs baseline: 1.6200x; 1.6200x over previous
import functools

import jax
import jax.numpy as jnp
from jax import lax
from jax.experimental import pallas as pl
from jax.experimental.pallas import tpu as pltpu
from jax.experimental.pallas import tpu_sc as plsc

_L = 12
_P = 100
_LEN = 5
_D = 768
_K = 5
_B = 32
_S = 2048

_G = 32
_OFFW = _L * _G

_NEG = -3.0e38


def _tc_body(x_ref, sk_ref, tk_ref,
             s_sim_ref, t_sim_ref, s_idx_ref, t_idx_ref,
             s_off_ref, t_off_ref, s_red_ref, t_red_ref):
    b = pl.program_id(0)
    xb = x_ref[0]
    mean = jnp.sum(xb, axis=0, keepdims=True) * (1.0 / _S)
    n2 = jnp.sum(mean * mean, axis=1, keepdims=True)
    xn = mean * lax.rsqrt(jnp.maximum(n2, 1e-12))

    @pl.when(b == 0)
    def _():
        s_red_ref[...] = jnp.zeros((1, 1), jnp.float32)
        t_red_ref[...] = jnp.zeros((1, 1), jnp.float32)

    ii = lax.broadcasted_iota(jnp.int32, (1, _P), 1)
    ii5 = lax.broadcasted_iota(jnp.int32, (1, _K), 1)
    pos = lax.broadcasted_iota(jnp.int32, (1, _OFFW), 1)
    lfield = pos // _G
    rem = pos - lfield * _G
    kfield = rem // _LEN
    jfield = rem - kfield * _LEN

    def route(k_ref, sim_ref, idx_ref, off_ref, red_ref):
        kk = k_ref[...]
        kn2 = jnp.sum(kk * kk, axis=1, keepdims=True)
        kn = kk * lax.rsqrt(jnp.maximum(kn2, 1e-12))
        sim = lax.dot_general(xn, kn, (((1,), (1,)), ((), ())),
                              preferred_element_type=jnp.float32)
        sim_ref[pl.ds(b, 1), :] = sim
        row = sim
        racc = jnp.zeros((1, 1), jnp.float32)
        ivec = jnp.zeros((1, _K), jnp.int32)
        isel = jnp.zeros((1, _OFFW), jnp.int32)
        for k in range(_K):
            mx = jnp.max(row, axis=1, keepdims=True)
            am = jnp.min(jnp.where(row == mx, ii, _P),
                         axis=1, keepdims=True)
            ivec = jnp.where(ii5 == k, am, ivec)
            isel = jnp.where(kfield == k, am, isel)
            racc = racc + mx
            row = jnp.where(ii == am, _NEG, row)
        idx_ref[pl.ds(b, 1), :] = ivec
        off_ref[pl.ds(b, 1), :] = lfield * (_P * _LEN) + isel * _LEN + jfield
        red_ref[...] = red_ref[...] + racc

        @pl.when(b == _B - 1)
        def _():
            red_ref[...] = red_ref[...] * (1.0 / _B)

    route(sk_ref, s_sim_ref, s_idx_ref, s_off_ref, s_red_ref)
    route(tk_ref, t_sim_ref, t_idx_ref, t_off_ref, t_red_ref)


_route_call = pl.pallas_call(
    _tc_body,
    grid=(_B,),
    in_specs=[
        pl.BlockSpec((1, _S, _D), lambda b: (b, 0, 0)),
        pl.BlockSpec((_P, _D), lambda b: (0, 0)),
        pl.BlockSpec((_P, _D), lambda b: (0, 0)),
    ],
    out_specs=[
        pl.BlockSpec((_B, _P), lambda b: (0, 0)),
        pl.BlockSpec((_B, _P), lambda b: (0, 0)),
        pl.BlockSpec((_B, _K), lambda b: (0, 0)),
        pl.BlockSpec((_B, _K), lambda b: (0, 0)),
        pl.BlockSpec((_B, _OFFW), lambda b: (0, 0)),
        pl.BlockSpec((_B, _OFFW), lambda b: (0, 0)),
        pl.BlockSpec((1, 1), lambda b: (0, 0)),
        pl.BlockSpec((1, 1), lambda b: (0, 0)),
    ],
    out_shape=[
        jax.ShapeDtypeStruct((_B, _P), jnp.float32),
        jax.ShapeDtypeStruct((_B, _P), jnp.float32),
        jax.ShapeDtypeStruct((_B, _K), jnp.int32),
        jax.ShapeDtypeStruct((_B, _K), jnp.int32),
        jax.ShapeDtypeStruct((_B, _OFFW), jnp.int32),
        jax.ShapeDtypeStruct((_B, _OFFW), jnp.int32),
        jax.ShapeDtypeStruct((1, 1), jnp.float32),
        jax.ShapeDtypeStruct((1, 1), jnp.float32),
    ],
)

_NC = 2
_NS = 16
_NW = _NC * _NS

_ROWS = _K * _LEN


@functools.cache
def _make_sc_gather():
    mesh = plsc.VectorSubcoreMesh(core_axis_name="c", subcore_axis_name="s",
                                  num_cores=_NC, num_subcores=_NS)

    @functools.partial(
        pl.kernel, mesh=mesh,
        out_type=[
            jax.ShapeDtypeStruct((_L * _B, _G, _D), jnp.float32),
            jax.ShapeDtypeStruct((_L * _B, _G, _D), jnp.float32),
        ],
        scratch_types=[
            pltpu.VMEM((_OFFW,), jnp.int32),
            pltpu.VMEM((_G, _D), jnp.float32),
            pltpu.VMEM((_G, _D), jnp.float32),
            pltpu.SemaphoreType.DMA,
            pltpu.SemaphoreType.DMA,
        ],
    )
    def _sc_gather(s_tab, t_tab, s_off, t_off, s_out, t_out,
                   off_v, buf_a, buf_b, sem_a, sem_b):
        b = lax.axis_index("s") * _NC + lax.axis_index("c")

        bufs = (buf_a, buf_b)
        sems = (sem_a, sem_b)

        def run(tab, off_hbm, out):
            pltpu.sync_copy(off_hbm.at[b], off_v)
            hs = [None] * _L
            hs[0] = pltpu.async_copy(tab.at[off_v.at[pl.ds(0, _G)]],
                                     bufs[0], sems[0])
            for lyr in range(_L):
                nxt = lyr + 1
                if nxt < _L:
                    hs[nxt] = pltpu.async_copy(
                        tab.at[off_v.at[pl.ds(nxt * _G, _G)]],
                        bufs[nxt % 2], sems[nxt % 2])
                hs[lyr].wait()
                pltpu.sync_copy(bufs[lyr % 2], out.at[lyr * _B + b])

        run(s_tab, s_off, s_out)
        run(t_tab, t_off, t_out)

    return _sc_gather


def kernel(x_embed, s_prompt, t_prompt, s_prompt_key, t_prompt_key):
    (s_sim, t_sim, s_idx, t_idx, s_off, t_off, s_red, t_red) = _route_call(
        x_embed, s_prompt_key, t_prompt_key)
    s_sc, t_sc = _make_sc_gather()(
        s_prompt.reshape(_L * _P * _LEN, _D),
        t_prompt.reshape(_L * _P * _LEN, _D),
        s_off, t_off)
    s_batched = s_sc.reshape(_L, _B, _G, _D)[:, :, :_ROWS, :]
    t_batched = t_sc.reshape(_L, _B, _G, _D)[:, :, :_ROWS, :]
    return (s_sim, t_sim, s_idx, t_idx, s_batched, t_batched,
            s_red.reshape(()), t_red.reshape(()))
